# baseline (speedup 1.0000x reference)
import functools

import jax
import jax.numpy as jnp
from jax import lax
from jax.experimental import pallas as pl
from jax.experimental.pallas import tpu as pltpu
from jax.experimental.pallas import tpu_sc as plsc

N_NODES = 10000
N_EDGES = 320000
D = 128
NC, NS, LANES = 2, 16, 16
NW = NC * NS
K = 64
NCH = 160
EPT = NCH * K
EPTA = 20480
NCHA = EPTA // K
N_PAD = 10240
HN = N_PAD // NC
AROWS = HN + 128
ARPT = AROWS // NS
BR = 512

_mesh = plsc.VectorSubcoreMesh(core_axis_name="c", subcore_axis_name="s",
                               num_cores=NC, num_subcores=NS)
_sc_params = pltpu.CompilerParams(needs_layout_passes=False,
                                  use_tc_tiling_on_sc=False)


@functools.partial(
    pl.kernel,
    out_type=jax.ShapeDtypeStruct((NW, N_PAD), jnp.float32),
    mesh=_mesh,
    compiler_params=_sc_params,
    scratch_types=[pltpu.VMEM((NCH, K), jnp.int32),
                   pltpu.VMEM((N_PAD,), jnp.float32)])
def _deg_kernel(dst_hbm, out_hbm, dst_v, hist_v):
    c = lax.axis_index("c")
    s = lax.axis_index("s")
    wid = s * NC + c
    pltpu.sync_copy(dst_hbm.at[wid], dst_v)
    zeros16 = jnp.zeros((LANES,), jnp.float32)

    def zero_body(i, _):
        hist_v[pl.ds(i * LANES, LANES)] = zeros16
        return 0
    lax.fori_loop(0, N_PAD // LANES, zero_body, 0)

    ones16 = jnp.ones((LANES,), jnp.float32)

    def hist_body(j, _):
        for k in range(K // LANES):
            idx = dst_v[j, pl.ds(k * LANES, LANES)]
            plsc.addupdate_scatter(hist_v, [idx], ones16)
        return 0
    lax.fori_loop(0, NCH, hist_body, 0)
    pltpu.sync_copy(hist_v, out_hbm.at[wid])


@functools.partial(
    pl.kernel,
    out_type=jax.ShapeDtypeStruct((NC, AROWS, D), jnp.float32),
    mesh=_mesh,
    compiler_params=_sc_params,
    scratch_types=[pltpu.VMEM((EPTA + K,), jnp.int32),
                   pltpu.VMEM((EPTA + K,), jnp.int32),
                   pltpu.VMEM((4, K, D), jnp.float32),
                   pltpu.VMEM((64, D), jnp.float32),
                   pltpu.VMEM_SHARED((AROWS, D), jnp.float32),
                   pltpu.SemaphoreType.DMA((4,)),
                   pltpu.SemaphoreType.DMA((4,))])
def _agg_kernel(y_hbm, src_hbm, dst_hbm, out_hbm,
                src_v, dst_v, rows_v, wb_v, acc_sh, gsems, ssems):
    c = lax.axis_index("c")
    s = lax.axis_index("s")
    zeros16 = jnp.zeros((LANES,), jnp.float32)

    def zero_body(i, _):
        for k in range(D // LANES):
            wb_v[i, pl.ds(k * LANES, LANES)] = zeros16
        return 0
    lax.fori_loop(0, 64, zero_body, 0)
    base = s * ARPT

    def zinit_body(i, _):
        pltpu.sync_copy(wb_v, acc_sh.at[pl.ds(s * 320 + i * 64, 64)])
        return 0
    lax.fori_loop(0, 5, zinit_body, 0)
    @pl.when(s == 0)
    def _zero_trash():
        pltpu.sync_copy(wb_v, acc_sh.at[pl.ds(HN, 64)])
        pltpu.sync_copy(wb_v, acc_sh.at[pl.ds(HN + 64, 64)])

    pltpu.sync_copy(src_hbm.at[s], src_v.at[pl.ds(0, EPTA)])
    pltpu.sync_copy(dst_hbm.at[s], dst_v.at[pl.ds(0, EPTA)])

    rbase = c * HN

    def compact_body(i, carry):
        off = carry
        sv = src_v[pl.ds(i * LANES, LANES)]
        dv = dst_v[pl.ds(i * LANES, LANES)]
        lv = dv - rbase
        mask = jnp.logical_and(lv >= 0, lv < HN)
        plsc.store_compressed(src_v.at[pl.ds(off, LANES)], sv, mask=mask)
        plsc.store_compressed(dst_v.at[pl.ds(off, LANES)], lv, mask=mask)
        return off + jnp.sum(mask.astype(jnp.int32))
    cnt = lax.fori_loop(0, EPTA // LANES, compact_body, 0)
    cnt_pad = ((cnt + K - 1) // K) * K
    pad_src = jnp.zeros((LANES,), jnp.int32)
    pad_dst = jnp.full((LANES,), HN, jnp.int32)

    def pad_body(k, _):
        src_v[pl.ds(cnt + k * LANES, LANES)] = pad_src
        dst_v[pl.ds(cnt + k * LANES, LANES)] = pad_dst
        return 0
    lax.fori_loop(0, K // LANES, pad_body, 0)
    nch = cnt_pad // K
    plsc.subcore_barrier()

    def chunk_body(j, _):
        b = lax.rem(j, 4)

        @pl.when(j >= 4)
        def _slot_free():
            pltpu.make_async_copy(
                y_hbm.at[pl.ds(0, K)], rows_v.at[b], ssems.at[b]).wait()

        @pl.when(j < nch)
        def _prefetch():
            pltpu.async_copy(
                y_hbm.at[src_v.at[pl.ds(j * K, K)]], rows_v.at[b],
                gsems.at[b])

        @pl.when(jnp.logical_and(j >= 2, j < nch + 2))
        def _consume():
            jm = j - 2
            bm = lax.rem(jm, 4)
            pltpu.make_async_copy(
                y_hbm.at[pl.ds(0, K)], rows_v.at[bm], gsems.at[bm]).wait()
            pltpu.async_copy(rows_v.at[bm],
                             acc_sh.at[dst_v.at[pl.ds(jm * K, K)]],
                             ssems.at[bm], add=True)
        return 0
    lax.fori_loop(0, nch + 4, chunk_body, 0)
    plsc.subcore_barrier()

    def wb64_body(i, _):
        off = base + i * 64
        pltpu.sync_copy(acc_sh.at[pl.ds(off, 64)], wb_v)
        pltpu.sync_copy(wb_v, out_hbm.at[c, pl.ds(off, 64)])
        return 0
    lax.fori_loop(0, 5, wb64_body, 0)
    off8 = base + 320
    pltpu.sync_copy(acc_sh.at[pl.ds(off8, 8)], wb_v.at[pl.ds(0, 8)])
    pltpu.sync_copy(wb_v.at[pl.ds(0, 8)], out_hbm.at[c, pl.ds(off8, 8)])


def _dinv_block(parts_ref):
    off = pl.multiple_of(pl.program_id(0) * BR, 128)
    deg = jnp.sum(parts_ref[:, pl.ds(off, BR)], axis=0) + 1.0
    return lax.rsqrt(deg)


def _mm_body(x_ref, w_ref, parts_ref, y_ref):
    dinv = _dinv_block(parts_ref)
    xw = jnp.dot(x_ref[...], w_ref[...], preferred_element_type=jnp.float32)
    y_ref[...] = xw * dinv[:, None]


def _fin_body(acc_ref, y_ref, parts_ref, b_ref, o_ref):
    dinv = _dinv_block(parts_ref)
    t = acc_ref[0] + y_ref[...]
    o_ref[...] = jnp.maximum(t * dinv[:, None] + b_ref[...], 0.0)


def kernel(x, edge_index, batch, W, b):
    del batch
    src = edge_index[0]
    dst = edge_index[1]
    pad = NW * EPT - N_EDGES
    src_p = jnp.concatenate([src, jnp.zeros((pad,), jnp.int32)])
    dst_p = jnp.concatenate([dst, jnp.full((pad,), N_NODES, jnp.int32)])

    parts = _deg_kernel(dst_p.reshape(NW, NCH, K))

    y = pl.pallas_call(
        _mm_body,
        grid=(pl.cdiv(N_NODES, BR),),
        in_specs=[pl.BlockSpec((BR, D), lambda i: (i, 0)),
                  pl.BlockSpec((D, D), lambda i: (0, 0)),
                  pl.BlockSpec((NW, N_PAD), lambda i: (0, 0))],
        out_specs=pl.BlockSpec((BR, D), lambda i: (i, 0)),
        out_shape=jax.ShapeDtypeStruct((N_NODES, D), jnp.float32),
    )(x, W, parts)

    accs = _agg_kernel(y, src_p.reshape(NS, EPTA), dst_p.reshape(NS, EPTA))

    nb = HN // BR
    out = pl.pallas_call(
        _fin_body,
        grid=(pl.cdiv(N_NODES, BR),),
        in_specs=[pl.BlockSpec((1, BR, D), lambda i: (i // nb, i % nb, 0)),
                  pl.BlockSpec((BR, D), lambda i: (i, 0)),
                  pl.BlockSpec((NW, N_PAD), lambda i: (0, 0)),
                  pl.BlockSpec((1, D), lambda i: (0, 0))],
        out_specs=pl.BlockSpec((BR, D), lambda i: (i, 0)),
        out_shape=jax.ShapeDtypeStruct((N_NODES, D), jnp.float32),
    )(accs, y, parts, b.reshape(1, D))
    return (out, None)


# speedup vs baseline: 1.7605x; 1.7605x over previous
import functools

import jax
import jax.numpy as jnp
from jax import lax
from jax.experimental import pallas as pl
from jax.experimental.pallas import tpu as pltpu
from jax.experimental.pallas import tpu_sc as plsc

N_NODES = 10000
N_EDGES = 320000
D = 128
DH = D // 2
NC, NS, LANES = 2, 16, 16
NW = NC * NS
K = 64
NCH = 160
EPT = NCH * K
NCHA = 320
N_PAD = 10240
RPT = N_PAD // NS
BR = 512

_mesh = plsc.VectorSubcoreMesh(core_axis_name="c", subcore_axis_name="s",
                               num_cores=NC, num_subcores=NS)
_sc_params = pltpu.CompilerParams(needs_layout_passes=False,
                                  use_tc_tiling_on_sc=False)


@functools.partial(
    pl.kernel,
    out_type=jax.ShapeDtypeStruct((NW, N_PAD), jnp.float32),
    mesh=_mesh,
    compiler_params=_sc_params,
    scratch_types=[pltpu.VMEM((NCH, K), jnp.int32),
                   pltpu.VMEM((N_PAD,), jnp.float32)])
def _deg_kernel(dst_hbm, out_hbm, dst_v, hist_v):
    c = lax.axis_index("c")
    s = lax.axis_index("s")
    wid = s * NC + c
    pltpu.sync_copy(dst_hbm.at[wid], dst_v)
    zeros16 = jnp.zeros((LANES,), jnp.float32)

    def zero_body(i, _):
        hist_v[pl.ds(i * LANES, LANES)] = zeros16
        return 0
    lax.fori_loop(0, N_PAD // LANES, zero_body, 0)

    ones16 = jnp.ones((LANES,), jnp.float32)

    def hist_body(j, _):
        for k in range(K // LANES):
            idx = dst_v[j, pl.ds(k * LANES, LANES)]
            plsc.addupdate_scatter(hist_v, [idx], ones16)
        return 0
    lax.fori_loop(0, NCH, hist_body, 0)
    pltpu.sync_copy(hist_v, out_hbm.at[wid])


@functools.partial(
    pl.kernel,
    out_type=jax.ShapeDtypeStruct((NC, N_PAD, DH), jnp.float32),
    mesh=_mesh,
    compiler_params=_sc_params,
    scratch_types=[pltpu.VMEM((NCHA, K), jnp.int32),
                   pltpu.VMEM((NCHA, K), jnp.int32),
                   pltpu.VMEM((8, K, DH), jnp.float32),
                   pltpu.VMEM((128, DH), jnp.float32),
                   pltpu.VMEM_SHARED((N_PAD, DH), jnp.float32),
                   pltpu.SemaphoreType.DMA((8,)),
                   pltpu.SemaphoreType.DMA((8,))])
def _agg_kernel(y2_hbm, src_hbm, dst_hbm, out_hbm,
                src_v, dst_v, rows_v, wb_v, acc_sh, gsems, ssems):
    c = lax.axis_index("c")
    s = lax.axis_index("s")
    zeros16 = jnp.zeros((LANES,), jnp.float32)

    def zero_body(i, _):
        for k in range(DH // LANES):
            wb_v[i, pl.ds(k * LANES, LANES)] = zeros16
        return 0
    lax.fori_loop(0, 128, zero_body, 0)
    base = s * RPT

    def zinit_body(i, _):
        pltpu.sync_copy(wb_v, acc_sh.at[pl.ds(base + i * 128, 128)])
        return 0
    lax.fori_loop(0, RPT // 128, zinit_body, 0)
    pltpu.sync_copy(src_hbm.at[s], src_v)
    pltpu.sync_copy(dst_hbm.at[s], dst_v)
    plsc.subcore_barrier()

    yc = y2_hbm.at[c]

    def chunk_body(j, _):
        b = lax.rem(j, 8)

        @pl.when(j >= 8)
        def _slot_free():
            pltpu.make_async_copy(
                yc.at[pl.ds(0, K)], rows_v.at[b], ssems.at[b]).wait()

        @pl.when(j < NCHA)
        def _prefetch():
            pltpu.async_copy(yc.at[src_v.at[j]], rows_v.at[b], gsems.at[b])

        @pl.when(jnp.logical_and(j >= 4, j < NCHA + 4))
        def _consume():
            jm = j - 4
            bm = lax.rem(jm, 8)
            pltpu.make_async_copy(
                yc.at[pl.ds(0, K)], rows_v.at[bm], gsems.at[bm]).wait()
            pltpu.async_copy(rows_v.at[bm], acc_sh.at[dst_v.at[jm]],
                             ssems.at[bm], add=True)
        return 0
    lax.fori_loop(0, NCHA + 8, chunk_body, 0)
    plsc.subcore_barrier()

    def wb_body(i, _):
        off = base + i * 128
        pltpu.sync_copy(acc_sh.at[pl.ds(off, 128)], wb_v)
        pltpu.sync_copy(wb_v, out_hbm.at[c, pl.ds(off, 128)])
        return 0
    lax.fori_loop(0, RPT // 128, wb_body, 0)


def _dinv_block(parts_ref):
    off = pl.multiple_of(pl.program_id(0) * BR, 128)
    deg = jnp.sum(parts_ref[:, pl.ds(off, BR)], axis=0) + 1.0
    return lax.rsqrt(deg)


def _mm_body(x_ref, w_ref, parts_ref, y2_ref):
    dinv = _dinv_block(parts_ref)
    xw = jnp.dot(x_ref[...], w_ref[...], preferred_element_type=jnp.float32)
    y = xw * dinv[:, None]
    y2_ref[0] = y[:, :DH]
    y2_ref[1] = y[:, DH:]


def _fin_body(acc_ref, y2_ref, parts_ref, b_ref, o_ref):
    dinv = _dinv_block(parts_ref)
    t = jnp.concatenate(
        [acc_ref[0] + y2_ref[0], acc_ref[1] + y2_ref[1]], axis=1)
    o_ref[...] = jnp.maximum(t * dinv[:, None] + b_ref[...], 0.0)


def kernel(x, edge_index, batch, W, b):
    del batch
    src = edge_index[0]
    dst = edge_index[1]
    pad = NW * EPT - N_EDGES
    src_p = jnp.concatenate([src, jnp.zeros((pad,), jnp.int32)])
    dst_p = jnp.concatenate([dst, jnp.full((pad,), N_NODES, jnp.int32)])

    parts = _deg_kernel(dst_p.reshape(NW, NCH, K))

    y2 = pl.pallas_call(
        _mm_body,
        grid=(pl.cdiv(N_NODES, BR),),
        in_specs=[pl.BlockSpec((BR, D), lambda i: (i, 0)),
                  pl.BlockSpec((D, D), lambda i: (0, 0)),
                  pl.BlockSpec((NW, N_PAD), lambda i: (0, 0))],
        out_specs=pl.BlockSpec((NC, BR, DH), lambda i: (0, i, 0)),
        out_shape=jax.ShapeDtypeStruct((NC, N_NODES, DH), jnp.float32),
    )(x, W, parts)

    accs = _agg_kernel(y2, src_p.reshape(NS, NCHA, K),
                       dst_p.reshape(NS, NCHA, K))

    out = pl.pallas_call(
        _fin_body,
        grid=(pl.cdiv(N_NODES, BR),),
        in_specs=[pl.BlockSpec((NC, BR, DH), lambda i: (0, i, 0)),
                  pl.BlockSpec((NC, BR, DH), lambda i: (0, i, 0)),
                  pl.BlockSpec((NW, N_PAD), lambda i: (0, 0)),
                  pl.BlockSpec((1, D), lambda i: (0, 0))],
        out_specs=pl.BlockSpec((BR, D), lambda i: (i, 0)),
        out_shape=jax.ShapeDtypeStruct((N_NODES, D), jnp.float32),
    )(accs, y2, parts, b.reshape(1, D))
    return (out, None)


# speedup vs baseline: 1.8184x; 1.0329x over previous
import functools

import jax
import jax.numpy as jnp
from jax import lax
from jax.experimental import pallas as pl
from jax.experimental.pallas import tpu as pltpu
from jax.experimental.pallas import tpu_sc as plsc

N_NODES = 10000
N_EDGES = 320000
D = 128
DH = D // 2
NC, NS, LANES = 2, 16, 16
NW = NC * NS
K = 64
NCH = 160
EPT = NCH * K
NCHA = 320
N_PAD = 10240
RPT = N_PAD // NS
BR = 512

_mesh = plsc.VectorSubcoreMesh(core_axis_name="c", subcore_axis_name="s",
                               num_cores=NC, num_subcores=NS)
_sc_params = pltpu.CompilerParams(needs_layout_passes=False,
                                  use_tc_tiling_on_sc=False)


@functools.partial(
    pl.kernel,
    out_type=jax.ShapeDtypeStruct((NW, N_PAD), jnp.float32),
    mesh=_mesh,
    compiler_params=_sc_params,
    scratch_types=[pltpu.VMEM((NCH, K), jnp.int32),
                   pltpu.VMEM((N_PAD,), jnp.float32)])
def _deg_kernel(dst_hbm, out_hbm, dst_v, hist_v):
    c = lax.axis_index("c")
    s = lax.axis_index("s")
    wid = s * NC + c
    pltpu.sync_copy(dst_hbm.at[wid], dst_v)
    zeros16 = jnp.zeros((LANES,), jnp.float32)

    def zero_body(i, _):
        hist_v[pl.ds(i * LANES, LANES)] = zeros16
        return 0
    lax.fori_loop(0, N_PAD // LANES, zero_body, 0)

    ones16 = jnp.ones((LANES,), jnp.float32)

    def hist_body(j, _):
        for k in range(K // LANES):
            idx = dst_v[j, pl.ds(k * LANES, LANES)]
            plsc.addupdate_scatter(hist_v, [idx], ones16)
        return 0
    lax.fori_loop(0, NCH, hist_body, 0)
    pltpu.sync_copy(hist_v, out_hbm.at[wid])


@functools.partial(
    pl.kernel,
    out_type=jax.ShapeDtypeStruct((NC, N_PAD, DH), jnp.float32),
    mesh=_mesh,
    compiler_params=_sc_params,
    scratch_types=[pltpu.VMEM((NCHA, K), jnp.int32),
                   pltpu.VMEM((NCHA, K), jnp.int32),
                   pltpu.VMEM((8, K, DH), jnp.float32),
                   pltpu.VMEM((128, DH), jnp.float32),
                   pltpu.VMEM_SHARED((N_PAD, DH), jnp.float32),
                   pltpu.SemaphoreType.DMA((8,)),
                   pltpu.SemaphoreType.DMA((8,))])
def _agg_kernel(y2_hbm, src_hbm, dst_hbm, out_hbm,
                src_v, dst_v, rows_v, wb_v, acc_sh, gsems, ssems):
    c = lax.axis_index("c")
    s = lax.axis_index("s")
    zeros16 = jnp.zeros((LANES,), jnp.float32)

    def zero_body(i, _):
        for k in range(DH // LANES):
            wb_v[i, pl.ds(k * LANES, LANES)] = zeros16
        return 0
    lax.fori_loop(0, 128, zero_body, 0)
    base = s * RPT

    def zinit_body(i, _):
        pltpu.sync_copy(wb_v, acc_sh.at[pl.ds(base + i * 128, 128)])
        return 0
    lax.fori_loop(0, RPT // 128, zinit_body, 0)
    pltpu.sync_copy(src_hbm.at[s], src_v)
    pltpu.sync_copy(dst_hbm.at[s], dst_v)
    plsc.subcore_barrier()

    yc = y2_hbm.at[c]

    def chunk_body(j, _):
        b = lax.rem(j, 8)

        @pl.when(j >= 8)
        def _slot_free():
            pltpu.make_async_copy(
                yc.at[pl.ds(0, K)], rows_v.at[b], ssems.at[b]).wait()

        @pl.when(j < NCHA)
        def _prefetch():
            pltpu.async_copy(yc.at[src_v.at[j]], rows_v.at[b], gsems.at[b])

        @pl.when(jnp.logical_and(j >= 4, j < NCHA + 4))
        def _consume():
            jm = j - 4
            bm = lax.rem(jm, 8)
            pltpu.make_async_copy(
                yc.at[pl.ds(0, K)], rows_v.at[bm], gsems.at[bm]).wait()
            pltpu.async_copy(rows_v.at[bm], acc_sh.at[dst_v.at[jm]],
                             ssems.at[bm], add=True)
        return 0
    lax.fori_loop(0, NCHA + 8, chunk_body, 0)
    plsc.subcore_barrier()

    def wb_body(i, _):
        off = base + i * 128
        pltpu.sync_copy(acc_sh.at[pl.ds(off, 128)], wb_v)
        pltpu.sync_copy(wb_v, out_hbm.at[c, pl.ds(off, 128)])
        return 0
    lax.fori_loop(0, RPT // 128, wb_body, 0)


def _dinv_block(parts_ref):
    deg = jnp.sum(parts_ref[...], axis=0) + 1.0
    return lax.rsqrt(deg)


def _mm_body(x_ref, w_ref, parts_ref, y2_ref):
    dinv = _dinv_block(parts_ref)
    xw = jnp.dot(x_ref[...], w_ref[...], preferred_element_type=jnp.float32)
    y = xw * dinv[:, None]
    y2_ref[0] = y[:, :DH]
    y2_ref[1] = y[:, DH:]


def _fin_body(acc_ref, y2_ref, parts_ref, b_ref, o_ref):
    dinv = _dinv_block(parts_ref)
    t = jnp.concatenate(
        [acc_ref[0] + y2_ref[0], acc_ref[1] + y2_ref[1]], axis=1)
    o_ref[...] = jnp.maximum(t * dinv[:, None] + b_ref[...], 0.0)


def kernel(x, edge_index, batch, W, b):
    del batch
    src = edge_index[0]
    dst = edge_index[1]
    pad = NW * EPT - N_EDGES
    src_p = jnp.concatenate([src, jnp.zeros((pad,), jnp.int32)])
    dst_p = jnp.concatenate([dst, jnp.full((pad,), N_NODES, jnp.int32)])

    parts = _deg_kernel(dst_p.reshape(NW, NCH, K))

    y2 = pl.pallas_call(
        _mm_body,
        grid=(pl.cdiv(N_NODES, BR),),
        in_specs=[pl.BlockSpec((BR, D), lambda i: (i, 0)),
                  pl.BlockSpec((D, D), lambda i: (0, 0)),
                  pl.BlockSpec((NW, BR), lambda i: (0, i))],
        out_specs=pl.BlockSpec((NC, BR, DH), lambda i: (0, i, 0)),
        out_shape=jax.ShapeDtypeStruct((NC, N_NODES, DH), jnp.float32),
    )(x, W, parts)

    accs = _agg_kernel(y2, src_p.reshape(NS, NCHA, K),
                       dst_p.reshape(NS, NCHA, K))

    out = pl.pallas_call(
        _fin_body,
        grid=(pl.cdiv(N_NODES, BR),),
        in_specs=[pl.BlockSpec((NC, BR, DH), lambda i: (0, i, 0)),
                  pl.BlockSpec((NC, BR, DH), lambda i: (0, i, 0)),
                  pl.BlockSpec((NW, BR), lambda i: (0, i)),
                  pl.BlockSpec((1, D), lambda i: (0, 0))],
        out_specs=pl.BlockSpec((BR, D), lambda i: (i, 0)),
        out_shape=jax.ShapeDtypeStruct((N_NODES, D), jnp.float32),
    )(accs, y2, parts, b.reshape(1, D))
    return (out, None)


# speedup vs baseline: 1.8535x; 1.0193x over previous
import functools

import jax
import jax.numpy as jnp
from jax import lax
from jax.experimental import pallas as pl
from jax.experimental.pallas import tpu as pltpu
from jax.experimental.pallas import tpu_sc as plsc

N_NODES = 10000
N_EDGES = 320000
D = 128
DH = D // 2
NC, NS, LANES = 2, 16, 16
NW = NC * NS
K = 64
NCH = 160
EPT = NCH * K
NCHA = 320
N_PAD = 10240
RPT = N_PAD // NS
BR = 1024

_mesh = plsc.VectorSubcoreMesh(core_axis_name="c", subcore_axis_name="s",
                               num_cores=NC, num_subcores=NS)
_sc_params = pltpu.CompilerParams(needs_layout_passes=False,
                                  use_tc_tiling_on_sc=False)


@functools.partial(
    pl.kernel,
    out_type=jax.ShapeDtypeStruct((NW, N_PAD), jnp.float32),
    mesh=_mesh,
    compiler_params=_sc_params,
    scratch_types=[pltpu.VMEM((NCH, K), jnp.int32),
                   pltpu.VMEM((N_PAD,), jnp.float32)])
def _deg_kernel(dst_hbm, out_hbm, dst_v, hist_v):
    c = lax.axis_index("c")
    s = lax.axis_index("s")
    wid = s * NC + c
    pltpu.sync_copy(dst_hbm.at[wid], dst_v)
    zeros16 = jnp.zeros((LANES,), jnp.float32)

    def zero_body(i, _):
        hist_v[pl.ds(i * LANES, LANES)] = zeros16
        return 0
    lax.fori_loop(0, N_PAD // LANES, zero_body, 0)

    ones16 = jnp.ones((LANES,), jnp.float32)

    def hist_body(j, _):
        for k in range(K // LANES):
            idx = dst_v[j, pl.ds(k * LANES, LANES)]
            plsc.addupdate_scatter(hist_v, [idx], ones16)
        return 0
    lax.fori_loop(0, NCH, hist_body, 0)
    pltpu.sync_copy(hist_v, out_hbm.at[wid])


@functools.partial(
    pl.kernel,
    out_type=jax.ShapeDtypeStruct((NC, N_PAD, DH), jnp.float32),
    mesh=_mesh,
    compiler_params=_sc_params,
    scratch_types=[pltpu.VMEM((NCHA, K), jnp.int32),
                   pltpu.VMEM((NCHA, K), jnp.int32),
                   pltpu.VMEM((8, K, DH), jnp.float32),
                   pltpu.VMEM((128, DH), jnp.float32),
                   pltpu.VMEM_SHARED((N_PAD, DH), jnp.float32),
                   pltpu.SemaphoreType.DMA((8,)),
                   pltpu.SemaphoreType.DMA((8,))])
def _agg_kernel(y2_hbm, src_hbm, dst_hbm, out_hbm,
                src_v, dst_v, rows_v, wb_v, acc_sh, gsems, ssems):
    c = lax.axis_index("c")
    s = lax.axis_index("s")
    zeros16 = jnp.zeros((LANES,), jnp.float32)

    def zero_body(i, _):
        for k in range(DH // LANES):
            wb_v[i, pl.ds(k * LANES, LANES)] = zeros16
        return 0
    lax.fori_loop(0, 128, zero_body, 0)
    base = s * RPT

    def zinit_body(i, _):
        pltpu.sync_copy(wb_v, acc_sh.at[pl.ds(base + i * 128, 128)])
        return 0
    lax.fori_loop(0, RPT // 128, zinit_body, 0)
    pltpu.sync_copy(src_hbm.at[s], src_v)
    pltpu.sync_copy(dst_hbm.at[s], dst_v)
    plsc.subcore_barrier()

    yc = y2_hbm.at[c]

    def chunk_body(j, _):
        b = lax.rem(j, 8)

        @pl.when(j >= 8)
        def _slot_free():
            pltpu.make_async_copy(
                yc.at[pl.ds(0, K)], rows_v.at[b], ssems.at[b]).wait()

        @pl.when(j < NCHA)
        def _prefetch():
            pltpu.async_copy(yc.at[src_v.at[j]], rows_v.at[b], gsems.at[b])

        @pl.when(jnp.logical_and(j >= 4, j < NCHA + 4))
        def _consume():
            jm = j - 4
            bm = lax.rem(jm, 8)
            pltpu.make_async_copy(
                yc.at[pl.ds(0, K)], rows_v.at[bm], gsems.at[bm]).wait()
            pltpu.async_copy(rows_v.at[bm], acc_sh.at[dst_v.at[jm]],
                             ssems.at[bm], add=True)
        return 0
    lax.fori_loop(0, NCHA + 8, chunk_body, 0)
    plsc.subcore_barrier()

    def wb_body(i, _):
        off = base + i * 128
        pltpu.sync_copy(acc_sh.at[pl.ds(off, 128)], wb_v)
        pltpu.sync_copy(wb_v, out_hbm.at[c, pl.ds(off, 128)])
        return 0
    lax.fori_loop(0, RPT // 128, wb_body, 0)


def _dinv_block(parts_ref):
    deg = jnp.sum(parts_ref[...], axis=0) + 1.0
    return lax.rsqrt(deg)


def _mm_body(x_ref, w_ref, parts_ref, y2_ref):
    dinv = _dinv_block(parts_ref)
    xw = jnp.dot(x_ref[...], w_ref[...], preferred_element_type=jnp.float32)
    y = xw * dinv[:, None]
    y2_ref[0] = y[:, :DH]
    y2_ref[1] = y[:, DH:]


def _fin_body(acc_ref, y2_ref, parts_ref, b_ref, o_ref):
    dinv = _dinv_block(parts_ref)
    t = jnp.concatenate(
        [acc_ref[0] + y2_ref[0], acc_ref[1] + y2_ref[1]], axis=1)
    o_ref[...] = jnp.maximum(t * dinv[:, None] + b_ref[...], 0.0)


def kernel(x, edge_index, batch, W, b):
    del batch
    src = edge_index[0]
    dst = edge_index[1]
    pad = NW * EPT - N_EDGES
    src_p = jnp.concatenate([src, jnp.zeros((pad,), jnp.int32)])
    dst_p = jnp.concatenate([dst, jnp.full((pad,), N_NODES, jnp.int32)])

    parts = _deg_kernel(dst_p.reshape(NW, NCH, K))

    y2 = pl.pallas_call(
        _mm_body,
        grid=(pl.cdiv(N_NODES, BR),),
        in_specs=[pl.BlockSpec((BR, D), lambda i: (i, 0)),
                  pl.BlockSpec((D, D), lambda i: (0, 0)),
                  pl.BlockSpec((NW, BR), lambda i: (0, i))],
        out_specs=pl.BlockSpec((NC, BR, DH), lambda i: (0, i, 0)),
        out_shape=jax.ShapeDtypeStruct((NC, N_NODES, DH), jnp.float32),
    )(x, W, parts)

    accs = _agg_kernel(y2, src_p.reshape(NS, NCHA, K),
                       dst_p.reshape(NS, NCHA, K))

    out = pl.pallas_call(
        _fin_body,
        grid=(pl.cdiv(N_NODES, BR),),
        in_specs=[pl.BlockSpec((NC, BR, DH), lambda i: (0, i, 0)),
                  pl.BlockSpec((NC, BR, DH), lambda i: (0, i, 0)),
                  pl.BlockSpec((NW, BR), lambda i: (0, i)),
                  pl.BlockSpec((1, D), lambda i: (0, 0))],
        out_specs=pl.BlockSpec((BR, D), lambda i: (i, 0)),
        out_shape=jax.ShapeDtypeStruct((N_NODES, D), jnp.float32),
    )(accs, y2, parts, b.reshape(1, D))
    return (out, None)


# speedup vs baseline: 1.8573x; 1.0020x over previous
import functools

import jax
import jax.numpy as jnp
from jax import lax
from jax.experimental import pallas as pl
from jax.experimental.pallas import tpu as pltpu
from jax.experimental.pallas import tpu_sc as plsc

N_NODES = 10000
N_EDGES = 320000
D = 128
DH = D // 2
NC, NS, LANES = 2, 16, 16
NW = NC * NS
K = 64
NCH = 160
EPT = NCH * K
NCHA = 320
N_PAD = 10240
RPT = N_PAD // NS
BR = 2048

_mesh = plsc.VectorSubcoreMesh(core_axis_name="c", subcore_axis_name="s",
                               num_cores=NC, num_subcores=NS)
_sc_params = pltpu.CompilerParams(needs_layout_passes=False,
                                  use_tc_tiling_on_sc=False)


@functools.partial(
    pl.kernel,
    out_type=jax.ShapeDtypeStruct((NW, N_PAD), jnp.float32),
    mesh=_mesh,
    compiler_params=_sc_params,
    scratch_types=[pltpu.VMEM((NCH, K), jnp.int32),
                   pltpu.VMEM((N_PAD,), jnp.float32)])
def _deg_kernel(dst_hbm, out_hbm, dst_v, hist_v):
    c = lax.axis_index("c")
    s = lax.axis_index("s")
    wid = s * NC + c
    pltpu.sync_copy(dst_hbm.at[wid], dst_v)
    zeros16 = jnp.zeros((LANES,), jnp.float32)

    def zero_body(i, _):
        hist_v[pl.ds(i * LANES, LANES)] = zeros16
        return 0
    lax.fori_loop(0, N_PAD // LANES, zero_body, 0)

    ones16 = jnp.ones((LANES,), jnp.float32)

    def hist_body(j, _):
        for k in range(K // LANES):
            idx = dst_v[j, pl.ds(k * LANES, LANES)]
            plsc.addupdate_scatter(hist_v, [idx], ones16)
        return 0
    lax.fori_loop(0, NCH, hist_body, 0)
    pltpu.sync_copy(hist_v, out_hbm.at[wid])


@functools.partial(
    pl.kernel,
    out_type=jax.ShapeDtypeStruct((NC, N_PAD, DH), jnp.float32),
    mesh=_mesh,
    compiler_params=_sc_params,
    scratch_types=[pltpu.VMEM((NCHA, K), jnp.int32),
                   pltpu.VMEM((NCHA, K), jnp.int32),
                   pltpu.VMEM((8, K, DH), jnp.float32),
                   pltpu.VMEM((128, DH), jnp.float32),
                   pltpu.VMEM_SHARED((N_PAD, DH), jnp.float32),
                   pltpu.SemaphoreType.DMA((8,)),
                   pltpu.SemaphoreType.DMA((8,))])
def _agg_kernel(y2_hbm, src_hbm, dst_hbm, out_hbm,
                src_v, dst_v, rows_v, wb_v, acc_sh, gsems, ssems):
    c = lax.axis_index("c")
    s = lax.axis_index("s")
    zeros16 = jnp.zeros((LANES,), jnp.float32)

    def zero_body(i, _):
        for k in range(DH // LANES):
            wb_v[i, pl.ds(k * LANES, LANES)] = zeros16
        return 0
    lax.fori_loop(0, 128, zero_body, 0)
    base = s * RPT

    def zinit_body(i, _):
        pltpu.sync_copy(wb_v, acc_sh.at[pl.ds(base + i * 128, 128)])
        return 0
    lax.fori_loop(0, RPT // 128, zinit_body, 0)
    pltpu.sync_copy(src_hbm.at[s], src_v)
    pltpu.sync_copy(dst_hbm.at[s], dst_v)
    plsc.subcore_barrier()

    yc = y2_hbm.at[c]

    def chunk_body(j, _):
        b = lax.rem(j, 8)

        @pl.when(j >= 8)
        def _slot_free():
            pltpu.make_async_copy(
                yc.at[pl.ds(0, K)], rows_v.at[b], ssems.at[b]).wait()

        @pl.when(j < NCHA)
        def _prefetch():
            pltpu.async_copy(yc.at[src_v.at[j]], rows_v.at[b], gsems.at[b])

        @pl.when(jnp.logical_and(j >= 4, j < NCHA + 4))
        def _consume():
            jm = j - 4
            bm = lax.rem(jm, 8)
            pltpu.make_async_copy(
                yc.at[pl.ds(0, K)], rows_v.at[bm], gsems.at[bm]).wait()
            pltpu.async_copy(rows_v.at[bm], acc_sh.at[dst_v.at[jm]],
                             ssems.at[bm], add=True)
        return 0
    lax.fori_loop(0, NCHA + 8, chunk_body, 0)
    plsc.subcore_barrier()

    def wb_body(i, _):
        off = base + i * 128
        pltpu.sync_copy(acc_sh.at[pl.ds(off, 128)], wb_v)
        pltpu.sync_copy(wb_v, out_hbm.at[c, pl.ds(off, 128)])
        return 0
    lax.fori_loop(0, RPT // 128, wb_body, 0)


def _dinv_block(parts_ref):
    deg = jnp.sum(parts_ref[...], axis=0) + 1.0
    return lax.rsqrt(deg)


def _mm_body(x_ref, w_ref, parts_ref, y2_ref):
    dinv = _dinv_block(parts_ref)
    xw = jnp.dot(x_ref[...], w_ref[...], preferred_element_type=jnp.float32)
    y = xw * dinv[:, None]
    y2_ref[0] = y[:, :DH]
    y2_ref[1] = y[:, DH:]


def _fin_body(acc_ref, y2_ref, parts_ref, b_ref, o_ref):
    dinv = _dinv_block(parts_ref)
    t = jnp.concatenate(
        [acc_ref[0] + y2_ref[0], acc_ref[1] + y2_ref[1]], axis=1)
    o_ref[...] = jnp.maximum(t * dinv[:, None] + b_ref[...], 0.0)


def kernel(x, edge_index, batch, W, b):
    del batch
    src = edge_index[0]
    dst = edge_index[1]
    pad = NW * EPT - N_EDGES
    src_p = jnp.concatenate([src, jnp.zeros((pad,), jnp.int32)])
    dst_p = jnp.concatenate([dst, jnp.full((pad,), N_NODES, jnp.int32)])

    parts = _deg_kernel(dst_p.reshape(NW, NCH, K))

    y2 = pl.pallas_call(
        _mm_body,
        grid=(pl.cdiv(N_NODES, BR),),
        in_specs=[pl.BlockSpec((BR, D), lambda i: (i, 0)),
                  pl.BlockSpec((D, D), lambda i: (0, 0)),
                  pl.BlockSpec((NW, BR), lambda i: (0, i))],
        out_specs=pl.BlockSpec((NC, BR, DH), lambda i: (0, i, 0)),
        out_shape=jax.ShapeDtypeStruct((NC, N_NODES, DH), jnp.float32),
    )(x, W, parts)

    accs = _agg_kernel(y2, src_p.reshape(NS, NCHA, K),
                       dst_p.reshape(NS, NCHA, K))

    out = pl.pallas_call(
        _fin_body,
        grid=(pl.cdiv(N_NODES, BR),),
        in_specs=[pl.BlockSpec((NC, BR, DH), lambda i: (0, i, 0)),
                  pl.BlockSpec((NC, BR, DH), lambda i: (0, i, 0)),
                  pl.BlockSpec((NW, BR), lambda i: (0, i)),
                  pl.BlockSpec((1, D), lambda i: (0, 0))],
        out_specs=pl.BlockSpec((BR, D), lambda i: (i, 0)),
        out_shape=jax.ShapeDtypeStruct((N_NODES, D), jnp.float32),
    )(accs, y2, parts, b.reshape(1, D))
    return (out, None)
